# baseline (speedup 1.0000x reference)
import jax
import jax.numpy as jnp
from jax.experimental import pallas as pl

HI = jax.lax.Precision.HIGHEST
BF = jnp.bfloat16

H = 256
D = 128
K = 1024
L = 8192


def _sd(a):
    return jnp.concatenate([jnp.zeros_like(a[:1]), a[:-1]], axis=0)


def _su(a):
    return jnp.concatenate([a[1:], jnp.zeros_like(a[:1])], axis=0)


def _mm(a, b):
    return jnp.dot(a, b, preferred_element_type=jnp.float32)


def _conv_s2k4(x_bf, w, b):
    T = x_bf.shape[0] // 2
    xr = x_bf.reshape(T, 2, x_bf.shape[1])
    A = xr[:, 0, :]
    B = xr[:, 1, :]
    return (((_mm(_sd(B), w[0]) + _mm(A, w[1])) + _mm(B, w[2]))
            + _mm(_su(A), w[3])) + b


def _conv_s1k3(x_bf, w, b):
    return ((_mm(_sd(x_bf), w[0]) + _mm(x_bf, w[1])) + _mm(_su(x_bf), w[2])) + b


def _deconv_s2k4(x_bf, w, b):
    ye = (_mm(_sd(x_bf), w[0]) + _mm(x_bf, w[2])) + b
    yo = (_mm(x_bf, w[1]) + _mm(_su(x_bf), w[3])) + b
    return ye, yo


def _vqvae_kernel(x_ref, w0_ref, b0_ref, w1_ref, b1_ref, w2_ref, b2_ref,
                  wz_ref, bz_ref, cb_ref, dwz_ref, dbz_ref,
                  dw0_ref, db0_ref, dw1_ref, db1_ref, dw2_ref, db2_ref,
                  dwo_ref, bo_ref, xh_ref, ze_ref, zq_ref):
    xr = x_ref[0]
    A = xr[:, 0:1].astype(BF)
    B = xr[:, 1:2].astype(BF)
    P = jnp.concatenate([_sd(B), A, B, _su(A)], axis=1)
    h0 = jax.nn.relu(_mm(P, w0_ref[...]) + b0_ref[...])
    h1 = jax.nn.relu(_conv_s2k4(h0.astype(BF), w1_ref[...], b1_ref[...]))
    h2 = jax.nn.relu(_conv_s2k4(h1.astype(BF), w2_ref[...], b2_ref[...]))
    z_e = _conv_s1k3(h2.astype(BF), wz_ref[...], bz_ref[...])
    ze_ref[0] = z_e

    cb = cb_ref[...]
    cbn = cb / (jnp.sqrt(jnp.sum(cb * cb, axis=1, keepdims=True)) + 1e-8)
    zn = z_e / (jnp.sqrt(jnp.sum(z_e * z_e, axis=1, keepdims=True)) + 1e-8)
    cbn_bf = cbn.astype(BF)
    sim = jax.lax.dot_general(zn.astype(BF), cbn_bf, (((1,), (1,)), ((), ())),
                              preferred_element_type=jnp.float32)
    idx = jnp.argmax(sim, axis=1)[:, None]
    oh = (jax.lax.broadcasted_iota(jnp.int32, (z_e.shape[0], K), 1)
          == idx).astype(BF)
    z_q = _mm(oh, cbn_bf)
    zq_ref[0] = z_q
    z_q_st = z_e + (z_q - z_e)

    g0 = jax.nn.relu(_conv_s1k3(z_q_st.astype(BF), dwz_ref[...], dbz_ref[...]))
    e, o = _deconv_s2k4(g0.astype(BF), dw0_ref[...], db0_ref[...])
    g1 = jax.nn.relu(jnp.stack([e, o], axis=1).reshape(2 * e.shape[0], H))
    e, o = _deconv_s2k4(g1.astype(BF), dw1_ref[...], db1_ref[...])
    g2 = jax.nn.relu(jnp.stack([e, o], axis=1).reshape(2 * e.shape[0], H))
    ge, go = _deconv_s2k4(g2.astype(BF), dw2_ref[...], db2_ref[...])
    ge = jax.nn.relu(ge).astype(BF).astype(jnp.float32)
    go = jax.nn.relu(go).astype(BF).astype(jnp.float32)
    wo = dwo_ref[...].astype(jnp.float32)
    xe = jnp.sum(_sd(go) * wo[0:1, :] + ge * wo[1:2, :] + go * wo[2:3, :],
                 axis=1, keepdims=True)
    xo = jnp.sum(ge * wo[0:1, :] + go * wo[1:2, :] + _su(ge) * wo[2:3, :],
                 axis=1, keepdims=True)
    xh_ref[0] = jnp.concatenate([xe, xo], axis=1) + bo_ref[0, 0]


@jax.jit
def kernel(x, enc_w0, enc_b0, enc_w1, enc_b1, enc_w2, enc_b2, enc_wz, enc_bz,
           codebook, dec_wz, dec_bz, dec_w0, dec_b0, dec_w1, dec_b1,
           dec_w2, dec_b2, dec_wout, dec_bout):
    Bn = x.shape[0]
    T0 = L // 2
    w0p = jnp.transpose(enc_w0[:, 0, :]).astype(BF)
    w1t = jnp.transpose(enc_w1, (2, 1, 0)).astype(BF)
    w2t = jnp.transpose(enc_w2, (2, 1, 0)).astype(BF)
    wzt = jnp.transpose(enc_wz, (2, 1, 0)).astype(BF)
    dwzt = jnp.transpose(dec_wz, (2, 1, 0)).astype(BF)
    dw0t = jnp.transpose(dec_w0, (2, 0, 1)).astype(BF)
    dw1t = jnp.transpose(dec_w1, (2, 0, 1)).astype(BF)
    dw2t = jnp.transpose(dec_w2, (2, 0, 1)).astype(BF)
    dwot = jnp.transpose(dec_wout[0]).astype(BF)
    xr = x.reshape(Bn, T0, 2)

    full = lambda *shape: pl.BlockSpec(shape, lambda b: (0,) * len(shape))
    r2 = lambda v: v.reshape(1, -1)
    xh, ze, zq = pl.pallas_call(
        _vqvae_kernel,
        grid=(Bn,),
        in_specs=[
            pl.BlockSpec((1, T0, 2), lambda b: (b, 0, 0)),
            full(4, H), full(1, H),
            full(4, H, H), full(1, H),
            full(4, H, H), full(1, H),
            full(3, H, D), full(1, D),
            full(K, D),
            full(3, D, H), full(1, H),
            full(4, H, H), full(1, H),
            full(4, H, H), full(1, H),
            full(4, H, H), full(1, H),
            full(3, H), full(1, 1),
        ],
        out_specs=[
            pl.BlockSpec((1, T0, 2), lambda b: (b, 0, 0)),
            pl.BlockSpec((1, K, D), lambda b: (b, 0, 0)),
            pl.BlockSpec((1, K, D), lambda b: (b, 0, 0)),
        ],
        out_shape=[
            jax.ShapeDtypeStruct((Bn, T0, 2), jnp.float32),
            jax.ShapeDtypeStruct((Bn, K, D), jnp.float32),
            jax.ShapeDtypeStruct((Bn, K, D), jnp.float32),
        ],
    )(xr, w0p, r2(enc_b0), w1t, r2(enc_b1), w2t, r2(enc_b2),
      wzt, r2(enc_bz), codebook, dwzt, r2(dec_bz),
      dw0t, r2(dec_b0), dw1t, r2(dec_b1), dw2t, r2(dec_b2),
      dwot, dec_bout.reshape(1, 1))
    return (xh.reshape(Bn, 1, L),
            jnp.transpose(ze, (0, 2, 1)),
            jnp.transpose(zq, (0, 2, 1)))


# speedup vs baseline: 1.0005x; 1.0005x over previous
import jax
import jax.numpy as jnp
from jax.experimental import pallas as pl
from jax.experimental.pallas import tpu as pltpu

HI = jax.lax.Precision.HIGHEST
BF = jnp.bfloat16

H = 256
D = 128
K = 1024
L = 8192


def _sd(a):
    return jnp.concatenate([jnp.zeros_like(a[:1]), a[:-1]], axis=0)


def _su(a):
    return jnp.concatenate([a[1:], jnp.zeros_like(a[:1])], axis=0)


def _mm(a, b):
    return jnp.dot(a, b, preferred_element_type=jnp.float32)


def _conv_s2k4(x_bf, w, b):
    T = x_bf.shape[0] // 2
    xr = x_bf.reshape(T, 2, x_bf.shape[1])
    A = xr[:, 0, :]
    B = xr[:, 1, :]
    return (((_mm(_sd(B), w[0]) + _mm(A, w[1])) + _mm(B, w[2]))
            + _mm(_su(A), w[3])) + b


def _conv_s1k3(x_bf, w, b):
    return ((_mm(_sd(x_bf), w[0]) + _mm(x_bf, w[1])) + _mm(_su(x_bf), w[2])) + b


def _deconv_s2k4(x_bf, w, b):
    ye = (_mm(_sd(x_bf), w[0]) + _mm(x_bf, w[2])) + b
    yo = (_mm(x_bf, w[1]) + _mm(_su(x_bf), w[3])) + b
    return ye, yo


def _vqvae_kernel(x_ref, w0_ref, b0_ref, w1_ref, b1_ref, w2_ref, b2_ref,
                  wz_ref, bz_ref, cb_ref, dwz_ref, dbz_ref,
                  dw0_ref, db0_ref, dw1_ref, db1_ref, dw2_ref, db2_ref,
                  dwo_ref, bo_ref, xh_ref, ze_ref, zq_ref):
    xr = x_ref[0]
    A = xr[:, 0:1].astype(BF)
    B = xr[:, 1:2].astype(BF)
    P = jnp.concatenate([_sd(B), A, B, _su(A)], axis=1)
    h0 = jax.nn.relu(_mm(P, w0_ref[...]) + b0_ref[...])
    h1 = jax.nn.relu(_conv_s2k4(h0.astype(BF), w1_ref[...], b1_ref[...]))
    h2 = jax.nn.relu(_conv_s2k4(h1.astype(BF), w2_ref[...], b2_ref[...]))
    z_e = _conv_s1k3(h2.astype(BF), wz_ref[...], bz_ref[...])
    ze_ref[0] = z_e

    cb = cb_ref[...]
    cbn = cb / (jnp.sqrt(jnp.sum(cb * cb, axis=1, keepdims=True)) + 1e-8)
    zn = z_e / (jnp.sqrt(jnp.sum(z_e * z_e, axis=1, keepdims=True)) + 1e-8)
    cbn_bf = cbn.astype(BF)
    sim = jax.lax.dot_general(zn.astype(BF), cbn_bf, (((1,), (1,)), ((), ())),
                              preferred_element_type=jnp.float32)
    idx = jnp.argmax(sim, axis=1)[:, None]
    oh = (jax.lax.broadcasted_iota(jnp.int32, (z_e.shape[0], K), 1)
          == idx).astype(BF)
    z_q = _mm(oh, cbn_bf)
    zq_ref[0] = z_q
    z_q_st = z_e + (z_q - z_e)

    g0 = jax.nn.relu(_conv_s1k3(z_q_st.astype(BF), dwz_ref[...], dbz_ref[...]))
    e, o = _deconv_s2k4(g0.astype(BF), dw0_ref[...], db0_ref[...])
    g1 = jax.nn.relu(jnp.stack([e, o], axis=1).reshape(2 * e.shape[0], H))
    e, o = _deconv_s2k4(g1.astype(BF), dw1_ref[...], db1_ref[...])
    g2 = jax.nn.relu(jnp.stack([e, o], axis=1).reshape(2 * e.shape[0], H))
    ge, go = _deconv_s2k4(g2.astype(BF), dw2_ref[...], db2_ref[...])
    ge = jax.nn.relu(ge).astype(BF).astype(jnp.float32)
    go = jax.nn.relu(go).astype(BF).astype(jnp.float32)
    wo = dwo_ref[...].astype(jnp.float32)
    xe = jnp.sum(_sd(go) * wo[0:1, :] + ge * wo[1:2, :] + go * wo[2:3, :],
                 axis=1, keepdims=True)
    xo = jnp.sum(ge * wo[0:1, :] + go * wo[1:2, :] + _su(ge) * wo[2:3, :],
                 axis=1, keepdims=True)
    xh_ref[0] = jnp.concatenate([xe, xo], axis=1) + bo_ref[0, 0]


@jax.jit
def kernel(x, enc_w0, enc_b0, enc_w1, enc_b1, enc_w2, enc_b2, enc_wz, enc_bz,
           codebook, dec_wz, dec_bz, dec_w0, dec_b0, dec_w1, dec_b1,
           dec_w2, dec_b2, dec_wout, dec_bout):
    Bn = x.shape[0]
    T0 = L // 2
    w0p = jnp.transpose(enc_w0[:, 0, :]).astype(BF)
    w1t = jnp.transpose(enc_w1, (2, 1, 0)).astype(BF)
    w2t = jnp.transpose(enc_w2, (2, 1, 0)).astype(BF)
    wzt = jnp.transpose(enc_wz, (2, 1, 0)).astype(BF)
    dwzt = jnp.transpose(dec_wz, (2, 1, 0)).astype(BF)
    dw0t = jnp.transpose(dec_w0, (2, 0, 1)).astype(BF)
    dw1t = jnp.transpose(dec_w1, (2, 0, 1)).astype(BF)
    dw2t = jnp.transpose(dec_w2, (2, 0, 1)).astype(BF)
    dwot = jnp.transpose(dec_wout[0]).astype(BF)
    xr = x.reshape(Bn, T0, 2)

    full = lambda *shape: pl.BlockSpec(shape, lambda b: (0,) * len(shape))
    r2 = lambda v: v.reshape(1, -1)
    xh, ze, zq = pl.pallas_call(
        _vqvae_kernel,
        grid=(Bn,),
        compiler_params=pltpu.CompilerParams(
            dimension_semantics=("parallel",)),
        in_specs=[
            pl.BlockSpec((1, T0, 2), lambda b: (b, 0, 0)),
            full(4, H), full(1, H),
            full(4, H, H), full(1, H),
            full(4, H, H), full(1, H),
            full(3, H, D), full(1, D),
            full(K, D),
            full(3, D, H), full(1, H),
            full(4, H, H), full(1, H),
            full(4, H, H), full(1, H),
            full(4, H, H), full(1, H),
            full(3, H), full(1, 1),
        ],
        out_specs=[
            pl.BlockSpec((1, T0, 2), lambda b: (b, 0, 0)),
            pl.BlockSpec((1, K, D), lambda b: (b, 0, 0)),
            pl.BlockSpec((1, K, D), lambda b: (b, 0, 0)),
        ],
        out_shape=[
            jax.ShapeDtypeStruct((Bn, T0, 2), jnp.float32),
            jax.ShapeDtypeStruct((Bn, K, D), jnp.float32),
            jax.ShapeDtypeStruct((Bn, K, D), jnp.float32),
        ],
    )(xr, w0p, r2(enc_b0), w1t, r2(enc_b1), w2t, r2(enc_b2),
      wzt, r2(enc_bz), codebook, dwzt, r2(dec_bz),
      dw0t, r2(dec_b0), dw1t, r2(dec_b1), dw2t, r2(dec_b2),
      dwot, dec_bout.reshape(1, 1))
    return (xh.reshape(Bn, 1, L),
            jnp.transpose(ze, (0, 2, 1)),
            jnp.transpose(zq, (0, 2, 1)))


# speedup vs baseline: 1.0123x; 1.0118x over previous
import jax
import jax.numpy as jnp
from jax.experimental import pallas as pl
from jax.experimental.pallas import tpu as pltpu

HI = jax.lax.Precision.HIGHEST
BF = jnp.bfloat16

H = 256
D = 128
K = 1024
L = 8192


def _sd(a):
    return jnp.concatenate([jnp.zeros_like(a[:1]), a[:-1]], axis=0)


def _su(a):
    return jnp.concatenate([a[1:], jnp.zeros_like(a[:1])], axis=0)


def _mm(a, b):
    return jnp.dot(a, b, preferred_element_type=jnp.float32)


def _conv_s2k4(x_bf, w, b):
    T = x_bf.shape[0] // 2
    xr = x_bf.reshape(T, 2, x_bf.shape[1])
    A = xr[:, 0, :]
    B = xr[:, 1, :]
    return (((_mm(_sd(B), w[0]) + _mm(A, w[1])) + _mm(B, w[2]))
            + _mm(_su(A), w[3])) + b


def _conv_s1k3(x_bf, w, b):
    return ((_mm(_sd(x_bf), w[0]) + _mm(x_bf, w[1])) + _mm(_su(x_bf), w[2])) + b


def _deconv_s2k4(x_bf, w, b):
    ye = (_mm(_sd(x_bf), w[0]) + _mm(x_bf, w[2])) + b
    yo = (_mm(x_bf, w[1]) + _mm(_su(x_bf), w[3])) + b
    return ye, yo


def _vqvae_kernel(x_ref, w0_ref, b0_ref, w1_ref, b1_ref, w2_ref, b2_ref,
                  wz_ref, bz_ref, cb_ref, dwz_ref, dbz_ref,
                  dw0_ref, db0_ref, dw1_ref, db1_ref, dw2_ref, db2_ref,
                  dwo_ref, bo_ref, xh_ref, ze_ref, zq_ref):
    xr = x_ref[0]
    A = xr[:, 0:1].astype(BF)
    B = xr[:, 1:2].astype(BF)
    P = jnp.concatenate([_sd(B), A, B, _su(A)], axis=1)
    h0 = jax.nn.relu(_mm(P, w0_ref[...]) + b0_ref[...])
    h1 = jax.nn.relu(_conv_s2k4(h0.astype(BF), w1_ref[...], b1_ref[...]))
    h2 = jax.nn.relu(_conv_s2k4(h1.astype(BF), w2_ref[...], b2_ref[...]))
    z_e = _conv_s1k3(h2.astype(BF), wz_ref[...], bz_ref[...])
    ze_ref[0] = z_e

    cb = cb_ref[...]
    cbn = cb / (jnp.sqrt(jnp.sum(cb * cb, axis=1, keepdims=True)) + 1e-8)
    zn = z_e / (jnp.sqrt(jnp.sum(z_e * z_e, axis=1, keepdims=True)) + 1e-8)
    cbn_bf = cbn.astype(BF)
    sim = jax.lax.dot_general(zn.astype(BF), cbn_bf, (((1,), (1,)), ((), ())),
                              preferred_element_type=jnp.float32)
    m = jnp.max(sim, axis=1, keepdims=True)
    iota = jax.lax.broadcasted_iota(jnp.int32, sim.shape, 1)
    mi = jnp.where(sim == m, iota, jnp.int32(K))
    idx = jnp.min(mi, axis=1, keepdims=True)
    oh = (mi == idx).astype(BF)
    z_q = _mm(oh, cbn_bf)
    zq_ref[0] = z_q
    z_q_st = z_e + (z_q - z_e)

    g0 = jax.nn.relu(_conv_s1k3(z_q_st.astype(BF), dwz_ref[...], dbz_ref[...]))
    e, o = _deconv_s2k4(g0.astype(BF), dw0_ref[...], db0_ref[...])
    g1 = jax.nn.relu(jnp.stack([e, o], axis=1).reshape(2 * e.shape[0], H))
    e, o = _deconv_s2k4(g1.astype(BF), dw1_ref[...], db1_ref[...])
    g2 = jax.nn.relu(jnp.stack([e, o], axis=1).reshape(2 * e.shape[0], H))
    ge, go = _deconv_s2k4(g2.astype(BF), dw2_ref[...], db2_ref[...])
    ge = jax.nn.relu(ge).astype(BF).astype(jnp.float32)
    go = jax.nn.relu(go).astype(BF).astype(jnp.float32)
    wo = dwo_ref[...].astype(jnp.float32)
    xe = jnp.sum(_sd(go) * wo[0:1, :] + ge * wo[1:2, :] + go * wo[2:3, :],
                 axis=1, keepdims=True)
    xo = jnp.sum(ge * wo[0:1, :] + go * wo[1:2, :] + _su(ge) * wo[2:3, :],
                 axis=1, keepdims=True)
    xh_ref[0] = jnp.concatenate([xe, xo], axis=1) + bo_ref[0, 0]


@jax.jit
def kernel(x, enc_w0, enc_b0, enc_w1, enc_b1, enc_w2, enc_b2, enc_wz, enc_bz,
           codebook, dec_wz, dec_bz, dec_w0, dec_b0, dec_w1, dec_b1,
           dec_w2, dec_b2, dec_wout, dec_bout):
    Bn = x.shape[0]
    T0 = L // 2
    w0p = jnp.transpose(enc_w0[:, 0, :]).astype(BF)
    w1t = jnp.transpose(enc_w1, (2, 1, 0)).astype(BF)
    w2t = jnp.transpose(enc_w2, (2, 1, 0)).astype(BF)
    wzt = jnp.transpose(enc_wz, (2, 1, 0)).astype(BF)
    dwzt = jnp.transpose(dec_wz, (2, 1, 0)).astype(BF)
    dw0t = jnp.transpose(dec_w0, (2, 0, 1)).astype(BF)
    dw1t = jnp.transpose(dec_w1, (2, 0, 1)).astype(BF)
    dw2t = jnp.transpose(dec_w2, (2, 0, 1)).astype(BF)
    dwot = jnp.transpose(dec_wout[0]).astype(BF)
    xr = x.reshape(Bn, T0, 2)

    full = lambda *shape: pl.BlockSpec(shape, lambda b: (0,) * len(shape))
    r2 = lambda v: v.reshape(1, -1)
    xh, ze, zq = pl.pallas_call(
        _vqvae_kernel,
        grid=(Bn,),
        compiler_params=pltpu.CompilerParams(
            dimension_semantics=("parallel",)),
        in_specs=[
            pl.BlockSpec((1, T0, 2), lambda b: (b, 0, 0)),
            full(4, H), full(1, H),
            full(4, H, H), full(1, H),
            full(4, H, H), full(1, H),
            full(3, H, D), full(1, D),
            full(K, D),
            full(3, D, H), full(1, H),
            full(4, H, H), full(1, H),
            full(4, H, H), full(1, H),
            full(4, H, H), full(1, H),
            full(3, H), full(1, 1),
        ],
        out_specs=[
            pl.BlockSpec((1, T0, 2), lambda b: (b, 0, 0)),
            pl.BlockSpec((1, K, D), lambda b: (b, 0, 0)),
            pl.BlockSpec((1, K, D), lambda b: (b, 0, 0)),
        ],
        out_shape=[
            jax.ShapeDtypeStruct((Bn, T0, 2), jnp.float32),
            jax.ShapeDtypeStruct((Bn, K, D), jnp.float32),
            jax.ShapeDtypeStruct((Bn, K, D), jnp.float32),
        ],
    )(xr, w0p, r2(enc_b0), w1t, r2(enc_b1), w2t, r2(enc_b2),
      wzt, r2(enc_bz), codebook, dwzt, r2(dec_bz),
      dw0t, r2(dec_b0), dw1t, r2(dec_b1), dw2t, r2(dec_b2),
      dwot, dec_bout.reshape(1, 1))
    return (xh.reshape(Bn, 1, L),
            jnp.transpose(ze, (0, 2, 1)),
            jnp.transpose(zq, (0, 2, 1)))
